# baseline (speedup 1.0000x reference)
import functools

import jax
import jax.numpy as jnp
from jax import lax
from jax.experimental import pallas as pl
from jax.experimental.pallas import tpu as pltpu
from jax.experimental.pallas import tpu_sc as plsc

N_NODES = 10000
N_MATCH = 4096
N_EDGES = 320000
NC = 2
NS = 16
L = 16

NP = 10240
RNG = NP // NS
EB = 20000
P1_PER_TILE = N_EDGES // NS
P2A_PER_TILE = N_EDGES // (NC * NS)
P2B_PER_TILE = N_EDGES // (NC * 2)
P2B_BLOCKS = P2B_PER_TILE // EB


def _dense_body(x_ref, w_ref, b_ref, out_ref):
    out_ref[...] = lax.dot_general(
        w_ref[...], x_ref[...], (((0,), (1,)), ((), ())),
        preferred_element_type=jnp.float32) + b_ref[...]


def _dense(x, wcat, bias):
    return pl.pallas_call(
        _dense_body,
        out_shape=jax.ShapeDtypeStruct((16, N_NODES), jnp.float32),
    )(x, wcat, bias)


def _combine_body(sp_ref, dt_ref, cnt_ref, out_ref):
    sp = sp_ref[...]
    s8 = sp[0, :8] + sp[0, 8:] + sp[1, :8] + sp[1, 8:]
    t = dt_ref[...] + s8
    z = jax.nn.sigmoid(t[:4])
    ht = jnp.tanh(t[4:])
    e = jnp.exp((1.0 - z) * ht)
    denom = jnp.sum(e * cnt_ref[...], axis=1, keepdims=True)
    out_ref[...] = e / denom


def _combine(sp, dt, cnt):
    return pl.pallas_call(
        _combine_body,
        out_shape=jax.ShapeDtypeStruct((4, N_NODES), jnp.float32),
    )(sp, dt, cnt)


@functools.partial(
    pl.kernel,
    out_type=(jax.ShapeDtypeStruct((NC, NS, N_NODES), jnp.float32),
              jax.ShapeDtypeStruct((N_NODES,), jnp.float32)),
    mesh=plsc.VectorSubcoreMesh(core_axis_name="c", subcore_axis_name="s"),
    scratch_types=[
        pltpu.VMEM((N_NODES,), jnp.float32),
        pltpu.VMEM((NP,), jnp.float32),
        pltpu.VMEM((NP,), jnp.float32),
        pltpu.VMEM((N_NODES,), jnp.float32),
        pltpu.VMEM((EB,), jnp.int32),
        pltpu.VMEM((EB,), jnp.int32),
        pltpu.VMEM((EB,), jnp.float32),
        pltpu.VMEM_SHARED((NS, NP), jnp.float32),
        pltpu.VMEM_SHARED((N_EDGES // NC,), jnp.float32),
        pltpu.SemaphoreType.DMA,
        pltpu.SemaphoreType.DMA,
        pltpu.SemaphoreType.DMA,
        pltpu.SemaphoreType.DMA,
    ],
    compiler_params=pltpu.CompilerParams(needs_layout_passes=False),
)
def _edge_kernel(src_hbm, dst_hbm, w_hbm, p_hbm, home_hbm, away_hbm,
                 spart_out, cnt_out,
                 p_loc, node, red_b, acc, src_b, dst_b, w_b, slots, normbuf,
                 sem0, sem1, sem2, sem3):
    c = lax.axis_index("c")
    s = lax.axis_index("s")
    zeros16 = jnp.zeros((L,), jnp.float32)

    off1 = s * P1_PER_TILE
    h_p = pltpu.async_copy(p_hbm.at[s % 8], p_loc, sem3)
    h_s = pltpu.async_copy(src_hbm.at[pl.ds(off1, P1_PER_TILE)], src_b, sem0)
    h_d = pltpu.async_copy(dst_hbm.at[pl.ds(off1, P1_PER_TILE)], dst_b, sem1)
    h_w = pltpu.async_copy(w_hbm.at[pl.ds(off1, P1_PER_TILE)], w_b, sem2)

    def _zero(ref, n):
        @plsc.parallel_loop(0, n // L, unroll=4)
        def zb(i):
            ref[pl.ds(i * L, L)] = zeros16

    _zero(node, NP)
    h_s.wait()
    h_d.wait()
    h_w.wait()

    @plsc.parallel_loop(0, P1_PER_TILE // L, unroll=8)
    def deg_it(i):
        sl = pl.ds(i * L, L)
        s16, d16, w16 = src_b[sl], dst_b[sl], w_b[sl]
        weff = jnp.where(s16 == d16, 0.0, w16)
        plsc.addupdate_scatter(node, [s16], weff)

    pltpu.sync_copy(node, slots.at[s])
    plsc.subcore_barrier()

    rbase = s * RNG
    for t0 in range(0, NS, 4):
        hs = [pltpu.async_copy(
                  slots.at[t0 + k].at[pl.ds(rbase, RNG)],
                  red_b.at[pl.ds((t0 + k) * RNG, RNG)], sem)
              for k, sem in ((0, sem0), (1, sem1), (2, sem2), (3, sem3))]
        for h in hs:
            h.wait()

    @plsc.parallel_loop(0, RNG // L, unroll=2)
    def red_it(i):
        sl = pl.ds(rbase + i * L, L)
        tot = red_b[pl.ds(i * L, L)]
        for t in range(1, NS):
            tot = tot + red_b[pl.ds(t * RNG + i * L, L)]
        d16 = tot
        bits = plsc.bitcast(d16, jnp.int32)
        bits = jnp.int32(0x5F3759DF) - (bits >> 1)
        y = plsc.bitcast(bits, jnp.float32)
        for _ in range(3):
            y = y * (1.5 - 0.5 * d16 * y * y)
        node[sl] = jnp.where(d16 > 0.0, y, 0.0)

    pltpu.sync_copy(node.at[pl.ds(rbase, RNG)], slots.at[0].at[pl.ds(rbase, RNG)])
    plsc.subcore_barrier()
    pltpu.sync_copy(slots.at[0], node)

    loc_a = s * P2A_PER_TILE
    off_a = c * (N_EDGES // NC) + loc_a
    h_s = pltpu.async_copy(src_hbm.at[pl.ds(off_a, P2A_PER_TILE)],
                           src_b.at[pl.ds(0, P2A_PER_TILE)], sem0)
    h_d = pltpu.async_copy(dst_hbm.at[pl.ds(off_a, P2A_PER_TILE)],
                           dst_b.at[pl.ds(0, P2A_PER_TILE)], sem1)
    h_w = pltpu.async_copy(w_hbm.at[pl.ds(off_a, P2A_PER_TILE)],
                           w_b.at[pl.ds(0, P2A_PER_TILE)], sem2)
    h_s.wait()
    h_d.wait()
    h_w.wait()

    @plsc.parallel_loop(0, P2A_PER_TILE // L, unroll=4)
    def norm_it(i):
        sl = pl.ds(i * L, L)
        s16, d16, w16 = src_b[sl], dst_b[sl], w_b[sl]
        weff = jnp.where(s16 == d16, 0.0, w16)
        ds16 = plsc.load_gather(node, [s16])
        dd16 = plsc.load_gather(node, [d16])
        w_b[sl] = -(ds16 * weff * dd16)
    pltpu.sync_copy(w_b.at[pl.ds(0, P2A_PER_TILE)],
                    normbuf.at[pl.ds(loc_a, P2A_PER_TILE)])

    @pl.when((c == 0) & (s == 0))
    def _cnt():
        _zero(acc, N_NODES)
        ones16 = jnp.full((L,), 1.0, jnp.float32)
        pltpu.sync_copy(home_hbm, src_b.at[pl.ds(0, N_MATCH)])
        pltpu.sync_copy(away_hbm, src_b.at[pl.ds(N_MATCH, N_MATCH)])

        @plsc.parallel_loop(0, 2 * N_MATCH // L, unroll=4)
        def hit(i):
            plsc.addupdate_scatter(acc, [src_b[pl.ds(i * L, L)]], ones16)
        pltpu.sync_copy(acc, cnt_out)

    _zero(acc, N_NODES)
    h_p.wait()
    plsc.subcore_barrier()

    h = s // 8

    def acc_block(b, _):
        loc = h * P2B_PER_TILE + b * EB
        off = c * (N_EDGES // NC) + loc
        h_s = pltpu.async_copy(src_hbm.at[pl.ds(off, EB)], src_b, sem0)
        h_d = pltpu.async_copy(dst_hbm.at[pl.ds(off, EB)], dst_b, sem1)
        h_n = pltpu.async_copy(normbuf.at[pl.ds(loc, EB)], w_b, sem2)
        h_s.wait()
        h_d.wait()
        h_n.wait()

        @plsc.parallel_loop(0, EB // L, unroll=8)
        def it(i):
            sl = pl.ds(i * L, L)
            s16, d16, n16 = src_b[sl], dst_b[sl], w_b[sl]
            p16 = plsc.load_gather(p_loc, [s16])
            plsc.addupdate_scatter(acc, [d16], n16 * p16)
        return 0
    lax.fori_loop(0, P2B_BLOCKS, acc_block, 0)

    pltpu.sync_copy(acc, spart_out.at[c].at[s])


@functools.partial(
    pl.kernel,
    out_type=jax.ShapeDtypeStruct((2 * N_MATCH, 4), jnp.float32),
    mesh=plsc.VectorSubcoreMesh(core_axis_name="c", subcore_axis_name="s"),
    scratch_types=[
        pltpu.VMEM((4, N_NODES), jnp.float32),
        pltpu.VMEM((256,), jnp.int32),
        pltpu.VMEM((256, 4), jnp.float32),
        pltpu.SemaphoreType.DMA,
        pltpu.SemaphoreType.DMA,
    ],
    compiler_params=pltpu.CompilerParams(needs_layout_passes=False),
)
def _gather_kernel(tab_hbm, idx_hbm, out_hbm, tab_loc, idx_v, rows_v,
                   sem0, sem1):
    c = lax.axis_index("c")
    s = lax.axis_index("s")
    base = (s * NC + c) * 256
    h_t = pltpu.async_copy(tab_hbm, tab_loc, sem0)
    h_i = pltpu.async_copy(idx_hbm.at[pl.ds(base, 256)], idx_v, sem1)
    h_t.wait()
    h_i.wait()
    iota = lax.iota(jnp.int32, L)

    @plsc.parallel_loop(0, 256 // L, unroll=2)
    def it(i):
        ha16 = idx_v[pl.ds(i * L, L)]
        row16 = iota + i * L
        for cc in range(4):
            cc16 = jnp.full((L,), cc, jnp.int32)
            g = plsc.load_gather(tab_loc, [cc16, ha16])
            plsc.store_scatter(rows_v, [row16, cc16], g)
    pltpu.sync_copy(rows_v, out_hbm.at[pl.ds(base, 256)])


def kernel(edge_index, home, away, edge_weight, embedding,
           W_xz, b_xz, W_hz, b_hz, W_xr, b_xr, W_hr, b_hr,
           W_xh, b_xh, W_hh, b_hh):
    x = embedding.astype(jnp.float32)
    src = edge_index[0].astype(jnp.int32)
    dst = edge_index[1].astype(jnp.int32)
    w = edge_weight.astype(jnp.float32)
    home32 = home.astype(jnp.int32)
    away32 = away.astype(jnp.int32)

    wcat = jnp.concatenate([W_xz[0], W_xh[0], W_xz[1], W_xh[1]], axis=1)
    bias = jnp.concatenate(
        [b_xz + b_hz, b_xh + b_hh, jnp.zeros((8,), jnp.float32)])[:, None]

    out1 = _dense(x, wcat, bias)
    dt = out1[0:8]
    pt = out1[8:16]

    sp, cnt = _edge_kernel(src, dst, w, pt, home32, away32)
    tab = _combine(sp, dt, cnt[None, :])
    ha = jnp.concatenate([home32, away32])
    return _gather_kernel(tab, ha)


# speedup vs baseline: 1.0917x; 1.0917x over previous
import functools

import jax
import jax.numpy as jnp
from jax import lax
from jax.experimental import pallas as pl
from jax.experimental.pallas import tpu as pltpu
from jax.experimental.pallas import tpu_sc as plsc

N_NODES = 10000
N_MATCH = 4096
N_EDGES = 320000
NC = 2
NS = 16
L = 16

NP = 10240
RNG = NP // NS
EB = 10000
P1_PER_TILE = N_EDGES // NS
P1_BLOCKS = P1_PER_TILE // EB
P2A_PER_TILE = N_EDGES // (NC * NS)
P2B_PER_TILE = N_EDGES // (NC * 2)
P2B_BLOCKS = P2B_PER_TILE // EB


def _dense_body(x_ref, w_ref, b_ref, out_ref):
    out_ref[...] = lax.dot_general(
        w_ref[...], x_ref[...], (((0,), (1,)), ((), ())),
        preferred_element_type=jnp.float32) + b_ref[...]


def _dense(x, wcat, bias):
    return pl.pallas_call(
        _dense_body,
        out_shape=jax.ShapeDtypeStruct((16, N_NODES), jnp.float32),
    )(x, wcat, bias)


def _combine_body(sp_ref, dt_ref, cnt_ref, out_ref):
    sp = sp_ref[...]
    s8 = sp[0, :8] + sp[0, 8:] + sp[1, :8] + sp[1, 8:]
    t = dt_ref[...] + s8
    z = jax.nn.sigmoid(t[:4])
    ht = jnp.tanh(t[4:])
    e = jnp.exp((1.0 - z) * ht)
    denom = jnp.sum(e * cnt_ref[...], axis=1, keepdims=True)
    out_ref[...] = e / denom


def _combine(sp, dt, cnt):
    return pl.pallas_call(
        _combine_body,
        out_shape=jax.ShapeDtypeStruct((4, N_NODES), jnp.float32),
    )(sp, dt, cnt)


@functools.partial(
    pl.kernel,
    out_type=(jax.ShapeDtypeStruct((NC, NS, N_NODES), jnp.float32),
              jax.ShapeDtypeStruct((N_NODES,), jnp.float32)),
    mesh=plsc.VectorSubcoreMesh(core_axis_name="c", subcore_axis_name="s"),
    scratch_types=[
        pltpu.VMEM((N_NODES,), jnp.float32),
        pltpu.VMEM((NP,), jnp.float32),
        pltpu.VMEM((NP,), jnp.float32),
        pltpu.VMEM((N_NODES,), jnp.float32),
        pltpu.VMEM((EB,), jnp.int32),
        pltpu.VMEM((EB,), jnp.int32),
        pltpu.VMEM((EB,), jnp.float32),
        pltpu.VMEM((EB,), jnp.int32),
        pltpu.VMEM((EB,), jnp.int32),
        pltpu.VMEM((EB,), jnp.float32),
        pltpu.VMEM_SHARED((NS, NP), jnp.float32),
        pltpu.VMEM_SHARED((N_EDGES // NC,), jnp.float32),
        pltpu.SemaphoreType.DMA,
        pltpu.SemaphoreType.DMA,
        pltpu.SemaphoreType.DMA,
        pltpu.SemaphoreType.DMA,
        pltpu.SemaphoreType.DMA,
        pltpu.SemaphoreType.DMA,
        pltpu.SemaphoreType.DMA,
        pltpu.SemaphoreType.DMA,
    ],
    compiler_params=pltpu.CompilerParams(needs_layout_passes=False),
)
def _edge_kernel(src_hbm, dst_hbm, w_hbm, p_hbm, home_hbm, away_hbm,
                 spart_out, cnt_out,
                 p_loc, node, red_b, acc, src0, dst0, w0, src1, dst1, w1,
                 slots, normbuf,
                 sem0, sem1, sem2, sem3, sem4, sem5, sem6, semp):
    c = lax.axis_index("c")
    s = lax.axis_index("s")
    zeros16 = jnp.zeros((L,), jnp.float32)
    bufsets = ((src0, dst0, w0, (sem0, sem1, sem2)),
               (src1, dst1, w1, (sem3, sem4, sem5)))

    def issue_edges(base, bufset):
        sb, db, wb, (ss, sd, sw) = bufset
        return (pltpu.async_copy(src_hbm.at[pl.ds(base, EB)], sb, ss),
                pltpu.async_copy(dst_hbm.at[pl.ds(base, EB)], db, sd),
                pltpu.async_copy(w_hbm.at[pl.ds(base, EB)], wb, sw))

    off1 = s * P1_PER_TILE
    h_p = pltpu.async_copy(p_hbm.at[s % 8], p_loc, semp)
    h1 = [issue_edges(off1 + k * EB, bufsets[k]) for k in range(P1_BLOCKS)]

    def _zero(ref, n):
        @plsc.parallel_loop(0, n // L, unroll=4)
        def zb(i):
            ref[pl.ds(i * L, L)] = zeros16

    _zero(node, NP)

    loc_a = s * P2A_PER_TILE
    off_a = c * (N_EDGES // NC) + loc_a
    for k in range(P1_BLOCKS):
        sb, db, wb, _ = bufsets[k]
        for h in h1[k]:
            h.wait()

        @plsc.parallel_loop(0, EB // L, unroll=4)
        def deg_it(i):
            sl = pl.ds(i * L, L)
            s16, d16, w16 = sb[sl], db[sl], wb[sl]
            weff = jnp.where(s16 == d16, 0.0, w16)
            plsc.addupdate_scatter(node, [s16], weff)

        if k == 0:
            h2a = issue_edges(off_a, bufsets[0])

    pltpu.sync_copy(node, slots.at[s])
    plsc.subcore_barrier()

    rbase = s * RNG
    for t0 in range(0, NS, 4):
        hs = [pltpu.async_copy(
                  slots.at[t0 + k].at[pl.ds(rbase, RNG)],
                  red_b.at[pl.ds((t0 + k) * RNG, RNG)], sem)
              for k, sem in ((0, sem3), (1, sem4), (2, sem5), (3, sem6))]
        for h in hs:
            h.wait()

    @plsc.parallel_loop(0, RNG // L, unroll=2)
    def red_it(i):
        sl = pl.ds(rbase + i * L, L)
        tot = red_b[pl.ds(i * L, L)]
        for t in range(1, NS):
            tot = tot + red_b[pl.ds(t * RNG + i * L, L)]
        d16 = tot
        bits = plsc.bitcast(d16, jnp.int32)
        bits = jnp.int32(0x5F3759DF) - (bits >> 1)
        y = plsc.bitcast(bits, jnp.float32)
        for _ in range(3):
            y = y * (1.5 - 0.5 * d16 * y * y)
        node[sl] = jnp.where(d16 > 0.0, y, 0.0)

    pltpu.sync_copy(node.at[pl.ds(rbase, RNG)], slots.at[0].at[pl.ds(rbase, RNG)])
    plsc.subcore_barrier()
    pltpu.sync_copy(slots.at[0], node)

    for h in h2a:
        h.wait()

    @plsc.parallel_loop(0, P2A_PER_TILE // L, unroll=4)
    def norm_it(i):
        sl = pl.ds(i * L, L)
        s16, d16, w16 = src0[sl], dst0[sl], w0[sl]
        weff = jnp.where(s16 == d16, 0.0, w16)
        ds16 = plsc.load_gather(node, [s16])
        dd16 = plsc.load_gather(node, [d16])
        w0[sl] = -(ds16 * weff * dd16)
    pltpu.sync_copy(w0, normbuf.at[pl.ds(loc_a, P2A_PER_TILE)])

    @pl.when((c == 0) & (s == 0))
    def _cnt():
        _zero(acc, N_NODES)
        ones16 = jnp.full((L,), 1.0, jnp.float32)
        pltpu.sync_copy(home_hbm, src0.at[pl.ds(0, N_MATCH)])
        pltpu.sync_copy(away_hbm, src0.at[pl.ds(N_MATCH, N_MATCH)])

        @plsc.parallel_loop(0, 2 * N_MATCH // L, unroll=4)
        def hit(i):
            plsc.addupdate_scatter(acc, [src0[pl.ds(i * L, L)]], ones16)
        pltpu.sync_copy(acc, cnt_out)

    _zero(acc, N_NODES)
    h_p.wait()
    plsc.subcore_barrier()

    half = s // 8

    def issue_2b(b):
        sb, db, wb, (ss, sd, sw) = bufsets[b % 2]
        loc = half * P2B_PER_TILE + b * EB
        off = c * (N_EDGES // NC) + loc
        return (pltpu.async_copy(src_hbm.at[pl.ds(off, EB)], sb, ss),
                pltpu.async_copy(dst_hbm.at[pl.ds(off, EB)], db, sd),
                pltpu.async_copy(normbuf.at[pl.ds(loc, EB)], wb, sw))

    pending = [issue_2b(0), issue_2b(1)]
    for b in range(P2B_BLOCKS):
        sb, db, wb, _ = bufsets[b % 2]
        for h in pending[b]:
            h.wait()

        @plsc.parallel_loop(0, EB // L, unroll=4)
        def it(i):
            sl = pl.ds(i * L, L)
            s16, d16, n16 = sb[sl], db[sl], wb[sl]
            p16 = plsc.load_gather(p_loc, [s16])
            plsc.addupdate_scatter(acc, [d16], n16 * p16)

        if b + 2 < P2B_BLOCKS:
            pending.append(issue_2b(b + 2))

    pltpu.sync_copy(acc, spart_out.at[c].at[s])


@functools.partial(
    pl.kernel,
    out_type=jax.ShapeDtypeStruct((2 * N_MATCH, 4), jnp.float32),
    mesh=plsc.VectorSubcoreMesh(core_axis_name="c", subcore_axis_name="s"),
    scratch_types=[
        pltpu.VMEM((4, N_NODES), jnp.float32),
        pltpu.VMEM((256,), jnp.int32),
        pltpu.VMEM((256, 4), jnp.float32),
        pltpu.SemaphoreType.DMA,
        pltpu.SemaphoreType.DMA,
    ],
    compiler_params=pltpu.CompilerParams(needs_layout_passes=False),
)
def _gather_kernel(tab_hbm, idx_hbm, out_hbm, tab_loc, idx_v, rows_v,
                   sem0, sem1):
    c = lax.axis_index("c")
    s = lax.axis_index("s")
    base = (s * NC + c) * 256
    h_t = pltpu.async_copy(tab_hbm, tab_loc, sem0)
    h_i = pltpu.async_copy(idx_hbm.at[pl.ds(base, 256)], idx_v, sem1)
    h_t.wait()
    h_i.wait()
    iota = lax.iota(jnp.int32, L)

    @plsc.parallel_loop(0, 256 // L, unroll=2)
    def it(i):
        ha16 = idx_v[pl.ds(i * L, L)]
        row16 = iota + i * L
        for cc in range(4):
            cc16 = jnp.full((L,), cc, jnp.int32)
            g = plsc.load_gather(tab_loc, [cc16, ha16])
            plsc.store_scatter(rows_v, [row16, cc16], g)
    pltpu.sync_copy(rows_v, out_hbm.at[pl.ds(base, 256)])


def kernel(edge_index, home, away, edge_weight, embedding,
           W_xz, b_xz, W_hz, b_hz, W_xr, b_xr, W_hr, b_hr,
           W_xh, b_xh, W_hh, b_hh):
    x = embedding.astype(jnp.float32)
    src = edge_index[0].astype(jnp.int32)
    dst = edge_index[1].astype(jnp.int32)
    w = edge_weight.astype(jnp.float32)
    home32 = home.astype(jnp.int32)
    away32 = away.astype(jnp.int32)

    wcat = jnp.concatenate([W_xz[0], W_xh[0], W_xz[1], W_xh[1]], axis=1)
    bias = jnp.concatenate(
        [b_xz + b_hz, b_xh + b_hh, jnp.zeros((8,), jnp.float32)])[:, None]

    out1 = _dense(x, wcat, bias)
    dt = out1[0:8]
    pt = out1[8:16]

    sp, cnt = _edge_kernel(src, dst, w, pt, home32, away32)
    tab = _combine(sp, dt, cnt[None, :])
    ha = jnp.concatenate([home32, away32])
    return _gather_kernel(tab, ha)


# speedup vs baseline: 1.1320x; 1.0369x over previous
import functools

import jax
import jax.numpy as jnp
from jax import lax
from jax.experimental import pallas as pl
from jax.experimental.pallas import tpu as pltpu
from jax.experimental.pallas import tpu_sc as plsc

N_NODES = 10000
N_MATCH = 4096
N_EDGES = 320000
NC = 2
NS = 16
L = 16

NP = 10240
RNG = NP // NS
EB = 10000
P1_PER_TILE = N_EDGES // NS
P1_BLOCKS = P1_PER_TILE // EB
P2A_PER_TILE = N_EDGES // (NC * NS)
P2B_PER_TILE = N_EDGES // (NC * 2)
P2B_BLOCKS = P2B_PER_TILE // EB


def _dense_body(x_ref, w_ref, b_ref, s_ref, d_ref, ew_ref,
                out_ref, pk_ref, we_ref):
    out_ref[...] = lax.dot_general(
        w_ref[...], x_ref[...], (((0,), (1,)), ((), ())),
        preferred_element_type=jnp.float32) + b_ref[...]
    s32 = s_ref[...]
    d32 = d_ref[...]
    pk_ref[...] = (s32 << 14) | d32
    we_ref[...] = jnp.where(s32 == d32, 0.0, ew_ref[...])


def _dense(x, wcat, bias, srcr, dstr, wr):
    er = srcr.shape
    return pl.pallas_call(
        _dense_body,
        out_shape=(jax.ShapeDtypeStruct((16, N_NODES), jnp.float32),
                   jax.ShapeDtypeStruct(er, jnp.int32),
                   jax.ShapeDtypeStruct(er, jnp.float32)),
    )(x, wcat, bias, srcr, dstr, wr)


def _combine_body(sp_ref, dt_ref, cnt_ref, out_ref):
    sp = sp_ref[...]
    s8 = sp[0, :8] + sp[0, 8:] + sp[1, :8] + sp[1, 8:]
    t = dt_ref[0:8] + s8
    z = jax.nn.sigmoid(t[:4])
    ht = jnp.tanh(t[4:])
    e = jnp.exp((1.0 - z) * ht)
    denom = jnp.sum(e * cnt_ref[...], axis=1, keepdims=True)
    out_ref[...] = e / denom


def _combine(sp, dt, cnt):
    return pl.pallas_call(
        _combine_body,
        out_shape=jax.ShapeDtypeStruct((4, N_NODES), jnp.float32),
    )(sp, dt, cnt)


@functools.partial(
    pl.kernel,
    out_type=(jax.ShapeDtypeStruct((NC, NS, N_NODES), jnp.float32),
              jax.ShapeDtypeStruct((N_NODES,), jnp.float32)),
    mesh=plsc.VectorSubcoreMesh(core_axis_name="c", subcore_axis_name="s"),
    scratch_types=[
        pltpu.VMEM((N_NODES,), jnp.float32),
        pltpu.VMEM((NP,), jnp.float32),
        pltpu.VMEM((NP,), jnp.float32),
        pltpu.VMEM((N_NODES,), jnp.float32),
        pltpu.VMEM((EB,), jnp.int32),
        pltpu.VMEM((EB,), jnp.float32),
        pltpu.VMEM((EB,), jnp.int32),
        pltpu.VMEM((EB,), jnp.float32),
        pltpu.VMEM_SHARED((NS, NP), jnp.float32),
        pltpu.VMEM_SHARED((N_EDGES // NC,), jnp.float32),
        pltpu.SemaphoreType.DMA,
        pltpu.SemaphoreType.DMA,
        pltpu.SemaphoreType.DMA,
        pltpu.SemaphoreType.DMA,
        pltpu.SemaphoreType.DMA,
        pltpu.SemaphoreType.DMA,
        pltpu.SemaphoreType.DMA,
    ],
    compiler_params=pltpu.CompilerParams(needs_layout_passes=False),
)
def _edge_kernel(pk_hbm, w_hbm, p_hbm, home_hbm, away_hbm,
                 spart_out, cnt_out,
                 p_loc, node, red_b, acc, pk0, w0, pk1, w1,
                 slots, normbuf,
                 sem0, sem1, sem2, sem3, sem4, sem5, semp):
    c = lax.axis_index("c")
    s = lax.axis_index("s")
    zeros16 = jnp.zeros((L,), jnp.float32)
    bufsets = ((pk0, w0, (sem0, sem1)),
               (pk1, w1, (sem2, sem3)))

    def issue_edges(base, bufset):
        pb, wb, (sp_, sw) = bufset
        return (pltpu.async_copy(pk_hbm.at[pl.ds(base, EB)], pb, sp_),
                pltpu.async_copy(w_hbm.at[pl.ds(base, EB)], wb, sw))

    off1 = s * P1_PER_TILE
    h_p = pltpu.async_copy(p_hbm.at[8 + (s % 8)], p_loc, semp)
    h1 = [issue_edges(off1 + k * EB, bufsets[k]) for k in range(P1_BLOCKS)]

    def _zero(ref, n):
        @plsc.parallel_loop(0, n // L, unroll=4)
        def zb(i):
            ref[pl.ds(i * L, L)] = zeros16

    _zero(node, NP)

    loc_a = s * P2A_PER_TILE
    off_a = c * (N_EDGES // NC) + loc_a
    for k in range(P1_BLOCKS):
        pb, wb, _ = bufsets[k]
        for h in h1[k]:
            h.wait()

        @plsc.parallel_loop(0, EB // L, unroll=4)
        def deg_it(i):
            sl = pl.ds(i * L, L)
            s16 = pb[sl] >> 14
            plsc.addupdate_scatter(node, [s16], wb[sl])

        if k == 0:
            h2a = issue_edges(off_a, bufsets[0])

    pltpu.sync_copy(node, slots.at[s])
    plsc.subcore_barrier()

    rbase = s * RNG
    for t0 in range(0, NS, 4):
        hs = [pltpu.async_copy(
                  slots.at[t0 + k].at[pl.ds(rbase, RNG)],
                  red_b.at[pl.ds((t0 + k) * RNG, RNG)], sem)
              for k, sem in ((0, sem2), (1, sem3), (2, sem4), (3, sem5))]
        for h in hs:
            h.wait()

    @plsc.parallel_loop(0, RNG // L, unroll=2)
    def red_it(i):
        sl = pl.ds(rbase + i * L, L)
        tot = red_b[pl.ds(i * L, L)]
        for t in range(1, NS):
            tot = tot + red_b[pl.ds(t * RNG + i * L, L)]
        d16 = tot
        bits = plsc.bitcast(d16, jnp.int32)
        bits = jnp.int32(0x5F3759DF) - (bits >> 1)
        y = plsc.bitcast(bits, jnp.float32)
        for _ in range(3):
            y = y * (1.5 - 0.5 * d16 * y * y)
        node[sl] = jnp.where(d16 > 0.0, y, 0.0)

    pltpu.sync_copy(node.at[pl.ds(rbase, RNG)], slots.at[0].at[pl.ds(rbase, RNG)])
    plsc.subcore_barrier()
    pltpu.sync_copy(slots.at[0], node)

    for h in h2a:
        h.wait()

    @plsc.parallel_loop(0, P2A_PER_TILE // L, unroll=4)
    def norm_it(i):
        sl = pl.ds(i * L, L)
        p16 = pk0[sl]
        ds16 = plsc.load_gather(node, [p16 >> 14])
        dd16 = plsc.load_gather(node, [p16 & 16383])
        w0[sl] = -(ds16 * w0[sl] * dd16)
    pltpu.sync_copy(w0, normbuf.at[pl.ds(loc_a, P2A_PER_TILE)])

    @pl.when((c == 0) & (s == 0))
    def _cnt():
        _zero(acc, N_NODES)
        ones16 = jnp.full((L,), 1.0, jnp.float32)
        pltpu.sync_copy(home_hbm, pk0.at[pl.ds(0, N_MATCH)])
        pltpu.sync_copy(away_hbm, pk0.at[pl.ds(N_MATCH, N_MATCH)])

        @plsc.parallel_loop(0, 2 * N_MATCH // L, unroll=4)
        def hit(i):
            plsc.addupdate_scatter(acc, [pk0[pl.ds(i * L, L)]], ones16)
        pltpu.sync_copy(acc, cnt_out)

    _zero(acc, N_NODES)
    h_p.wait()
    plsc.subcore_barrier()

    half = s // 8

    def issue_2b(b):
        pb, wb, (sp_, sw) = bufsets[b % 2]
        loc = half * P2B_PER_TILE + b * EB
        off = c * (N_EDGES // NC) + loc
        return (pltpu.async_copy(pk_hbm.at[pl.ds(off, EB)], pb, sp_),
                pltpu.async_copy(normbuf.at[pl.ds(loc, EB)], wb, sw))

    pending = [issue_2b(0), issue_2b(1)]
    for b in range(P2B_BLOCKS):
        pb, wb, _ = bufsets[b % 2]
        for h in pending[b]:
            h.wait()

        @plsc.parallel_loop(0, EB // L, unroll=4)
        def it(i):
            sl = pl.ds(i * L, L)
            pk16, n16 = pb[sl], wb[sl]
            p16 = plsc.load_gather(p_loc, [pk16 >> 14])
            plsc.addupdate_scatter(acc, [pk16 & 16383], n16 * p16)

        if b + 2 < P2B_BLOCKS:
            pending.append(issue_2b(b + 2))

    pltpu.sync_copy(acc, spart_out.at[c].at[s])


@functools.partial(
    pl.kernel,
    out_type=jax.ShapeDtypeStruct((2 * N_MATCH, 4), jnp.float32),
    mesh=plsc.VectorSubcoreMesh(core_axis_name="c", subcore_axis_name="s"),
    scratch_types=[
        pltpu.VMEM((4, N_NODES), jnp.float32),
        pltpu.VMEM((256,), jnp.int32),
        pltpu.VMEM((256, 4), jnp.float32),
        pltpu.SemaphoreType.DMA,
        pltpu.SemaphoreType.DMA,
    ],
    compiler_params=pltpu.CompilerParams(needs_layout_passes=False),
)
def _gather_kernel(tab_hbm, home_hbm, away_hbm, out_hbm, tab_loc, idx_v,
                   rows_v, sem0, sem1):
    c = lax.axis_index("c")
    s = lax.axis_index("s")
    w = s * NC + c
    base = w * 256
    h_t = pltpu.async_copy(tab_hbm, tab_loc, sem0)

    @pl.when(w < 16)
    def _ih():
        pltpu.async_copy(home_hbm.at[pl.ds(base, 256)], idx_v, sem1).wait()

    @pl.when(w >= 16)
    def _ia():
        pltpu.async_copy(
            away_hbm.at[pl.ds(base - N_MATCH, 256)], idx_v, sem1).wait()

    h_t.wait()
    iota = lax.iota(jnp.int32, L)

    @plsc.parallel_loop(0, 256 // L, unroll=2)
    def it(i):
        ha16 = idx_v[pl.ds(i * L, L)]
        row16 = iota + i * L
        for cc in range(4):
            cc16 = jnp.full((L,), cc, jnp.int32)
            g = plsc.load_gather(tab_loc, [cc16, ha16])
            plsc.store_scatter(rows_v, [row16, cc16], g)
    pltpu.sync_copy(rows_v, out_hbm.at[pl.ds(base, 256)])


def kernel(edge_index, home, away, edge_weight, embedding,
           W_xz, b_xz, W_hz, b_hz, W_xr, b_xr, W_hr, b_hr,
           W_xh, b_xh, W_hh, b_hh):
    x = embedding.astype(jnp.float32)
    src = edge_index[0].astype(jnp.int32)
    dst = edge_index[1].astype(jnp.int32)
    w = edge_weight.astype(jnp.float32)
    home32 = home.astype(jnp.int32)
    away32 = away.astype(jnp.int32)

    wcat = jnp.concatenate([W_xz[0], W_xh[0], W_xz[1], W_xh[1]], axis=1)
    bias = jnp.concatenate(
        [b_xz + b_hz, b_xh + b_hh, jnp.zeros((8,), jnp.float32)])[:, None]

    out1, pkr, wer = _dense(x, wcat, bias,
                            src.reshape(-1, 128), dst.reshape(-1, 128),
                            w.reshape(-1, 128))
    packed = pkr.reshape(-1)
    weff = wer.reshape(-1)

    sp, cnt = _edge_kernel(packed, weff, out1, home32, away32)
    tab = _combine(sp, out1, cnt[None, :])
    return _gather_kernel(tab, home32, away32)
